# baseline (speedup 1.0000x reference)
import functools

import jax
import jax.numpy as jnp
from jax import lax
from jax.experimental import pallas as pl
from jax.experimental.pallas import tpu as pltpu
from jax.experimental.pallas import tpu_sc as plsc

_LANES = 16


def kernel(inputs, emb):
    B, H = inputs.shape
    V, D = emb.shape
    NC, NS = 2, 16
    NW = NC * NS
    BPW = B // NW
    SPLIT = D // _LANES

    table = emb.reshape(V * SPLIT, _LANES)

    mesh = plsc.VectorSubcoreMesh(
        core_axis_name="c", subcore_axis_name="s", num_cores=NC, num_subcores=NS
    )

    @functools.partial(
        pl.kernel,
        out_type=jax.ShapeDtypeStruct((B, D), jnp.float32),
        mesh=mesh,
        scratch_types=[
            pltpu.VMEM((BPW, H), jnp.int32),
            pltpu.VMEM((H, BPW * SPLIT), jnp.int32),
            pltpu.VMEM((BPW * SPLIT, _LANES), jnp.float32),
            pltpu.VMEM((BPW, D), jnp.float32),
            pltpu.SemaphoreType.DMA,
        ],
        compiler_params=pltpu.CompilerParams(
            use_tc_tiling_on_sc=False, needs_layout_passes=False
        ),
    )
    def body(idx_hbm, tab_hbm, out_hbm, raw_v, idx_v, acc_v, acc2_v, sem):
        wid = lax.axis_index("s") * NC + lax.axis_index("c")
        pltpu.sync_copy(idx_hbm.at[pl.ds(wid * BPW, BPW)], raw_v)

        def zero_row(i, carry):
            acc_v[i, pl.ds(0, _LANES)] = jnp.zeros((_LANES,), jnp.float32)
            return carry

        lax.fori_loop(0, BPW * SPLIT, zero_row, 0)

        lane = lax.iota(jnp.int32, _LANES)

        def column(j, carry):
            col = jnp.full((_LANES,), j, jnp.int32)
            row_j = jnp.full((_LANES,), j, jnp.int32)

            def chunk(c, carry2):
                r = plsc.load_gather(raw_v, [c * _LANES + lane, col])
                pos = c * (_LANES * SPLIT) + SPLIT * lane
                for s in range(SPLIT):
                    plsc.store_scatter(idx_v, [row_j, pos + s], SPLIT * r + s)
                return carry2

            lax.fori_loop(0, BPW // _LANES, chunk, 0)
            pltpu.async_copy(tab_hbm.at[idx_v.at[j]], acc_v, sem, add=True)
            return carry

        lax.fori_loop(0, H, column, 0)

        def drain(j, carry):
            pltpu.make_async_copy(tab_hbm.at[idx_v.at[j]], acc_v, sem).wait()
            return carry

        lax.fori_loop(0, H, drain, 0)

        def repack(i, carry):
            v = acc_v[i, pl.ds(0, _LANES)]
            acc2_v[i // SPLIT, pl.ds((i % SPLIT) * _LANES, _LANES)] = v
            return carry

        lax.fori_loop(0, BPW * SPLIT, repack, 0)

        pltpu.sync_copy(acc2_v, out_hbm.at[pl.ds(wid * BPW, BPW)])

    return body(inputs, table)
